# baseline (speedup 1.0000x reference)
import functools

import jax
import jax.numpy as jnp
from jax import lax
from jax.experimental import pallas as pl
from jax.experimental.pallas import tpu as pltpu
from jax.experimental.pallas import tpu_sc as plsc

V = 1000
N = 51200
NC, NS, L = 2, 16, 16
NW = NC * NS
RPW = N // NW
CHL = 64
NCHL = RPW // CHL
TW0, TW1 = 1536, 1664
TWMAX = TW1
NSLAB = V // 8


def _lse_body(tab_ref, out_ref):
    x = tab_ref[...]
    m = jnp.max(x, axis=1, keepdims=True)
    s = jnp.sum(jnp.exp(x - m), axis=1, keepdims=True)
    out_ref[...] = jnp.log(s) + m


def _row_lse(table):
    return pl.pallas_call(
        _lse_body,
        out_shape=jax.ShapeDtypeStruct((V, 1), jnp.float32),
    )(table)


def _fin_body(p_ref, out_ref):
    out_ref[...] = jnp.full((1, 1), jnp.sum(p_ref[...]) * (1.0 / N),
                            dtype=jnp.float32)


def _finalize(partials):
    return pl.pallas_call(
        _fin_body,
        out_shape=jax.ShapeDtypeStruct((1, 1), jnp.float32),
    )(partials)


def _sc_body(tabTf_hbm, tabf_hbm, lse_hbm, in_hbm, tg_hbm,
             outT_hbm, part_hbm,
             idxg_v, idx_v, tgt_v, fidx_v, lse_v,
             slab0, slab1, ob0, ob1, telem_v, acc_v,
             g0, g1, s0, s1, t3):
    wid = lax.axis_index("s") * NC + lax.axis_index("c")

    base = wid * RPW
    pltpu.sync_copy(in_hbm.at[pl.ds(base, RPW)], idx_v)
    pltpu.sync_copy(tg_hbm.at[pl.ds(base, RPW)], tgt_v)
    pltpu.sync_copy(lse_hbm, lse_v)
    acc_v[...] = jnp.zeros((L,), jnp.float32)

    def fx(i, carry):
        o = i * L
        fidx_v[pl.ds(o, L)] = idx_v[pl.ds(o, L)] * V + tgt_v[pl.ds(o, L)]
        return carry

    lax.fori_loop(0, RPW // L, fx, 0)

    def lchunk(c, carry):
        co = c * CHL
        pltpu.async_copy(
            tabf_hbm.at[fidx_v.at[pl.ds(co, CHL)]], telem_v, t3).wait()
        for g in range(CHL // L):
            ids = idx_v[pl.ds(co + g * L, L)]
            lse_g = plsc.load_gather(lse_v, [ids])
            acc_v[...] = acc_v[...] + (lse_g - telem_v[pl.ds(g * L, L)])
        return carry

    lax.fori_loop(0, NCHL, lchunk, 0)
    pltpu.sync_copy(acc_v, part_hbm.at[wid])

    c0 = jnp.where(wid < 16, wid * TW0, 16 * TW0 + (wid - 16) * TW1)
    pltpu.sync_copy(in_hbm.at[pl.ds(c0, TWMAX)], idxg_v)

    def span_loop(tw):
        nt = tw // 128
        bufs = ((slab0, ob0, g0, s0), (slab1, ob1, g1, s1))

        def fetch(s, slab, gsem):
            pltpu.async_copy(tabTf_hbm.at[pl.ds(s * 8000, 8000)],
                             slab, gsem)

        def store(s, ob, ssem):
            return pltpu.async_copy(
                ob.at[:, pl.ds(0, tw)],
                outT_hbm.at[pl.ds(s * 8, 8), pl.ds(c0, tw)], ssem)

        fetch(0, slab0, g0)
        fetch(1, slab1, g1)

        def do_slab(s, slab, ob, gsem, ssem, first):
            pltpu.make_async_copy(
                tabTf_hbm.at[pl.ds(0, 8000)], slab, gsem).wait()
            if not first:
                pltpu.make_async_copy(
                    ob.at[:, pl.ds(0, tw)],
                    outT_hbm.at[pl.ds(0, 8), pl.ds(c0, tw)], ssem).wait()

            def tile_body(t, carry):
                for k in range(8):
                    off = t * 128 + k * 16
                    ids = idxg_v[pl.ds(off, L)]
                    for vv in range(8):
                        vals = plsc.load_gather(slab, [ids + vv * V])
                        ob[vv, pl.ds(off, L)] = vals
                return carry

            lax.fori_loop(0, nt, tile_body, 0)
            store(s, ob, ssem)

            @pl.when(s + 2 < NSLAB)
            def _():
                fetch(s + 2, slab, gsem)

        def pair(s2, carry):
            s = s2 * 2
            do_slab(s, slab0, ob0, g0, s0, False)
            do_slab(s + 1, slab1, ob1, g1, s1, False)
            return carry

        do_slab(0, slab0, ob0, g0, s0, True)
        do_slab(1, slab1, ob1, g1, s1, True)
        lax.fori_loop(1, (NSLAB - 1) // 2, pair, 0)
        do_slab(NSLAB - 1, slab0, ob0, g0, s0, False)
        pltpu.make_async_copy(
            ob0.at[:, pl.ds(0, tw)],
            outT_hbm.at[pl.ds(0, 8), pl.ds(c0, tw)], s0).wait()
        pltpu.make_async_copy(
            ob1.at[:, pl.ds(0, tw)],
            outT_hbm.at[pl.ds(0, 8), pl.ds(c0, tw)], s1).wait()

    @pl.when(wid < 16)
    def _():
        span_loop(TW0)

    @pl.when(wid >= 16)
    def _():
        span_loop(TW1)


@functools.partial(
    pl.kernel,
    out_type=(
        jax.ShapeDtypeStruct((V, N), jnp.float32),
        jax.ShapeDtypeStruct((NW, L), jnp.float32),
    ),
    mesh=plsc.VectorSubcoreMesh(core_axis_name="c", subcore_axis_name="s"),
    compiler_params=pltpu.CompilerParams(use_tc_tiling_on_sc=True,
                                         needs_layout_passes=False),
    scratch_types=[
        pltpu.VMEM((TWMAX,), jnp.int32),
        pltpu.VMEM((RPW,), jnp.int32),
        pltpu.VMEM((RPW,), jnp.int32),
        pltpu.VMEM((RPW,), jnp.int32),
        pltpu.VMEM((V,), jnp.float32),
        pltpu.VMEM((8000,), jnp.float32),
        pltpu.VMEM((8000,), jnp.float32),
        pltpu.VMEM((8, TWMAX), jnp.float32),
        pltpu.VMEM((8, TWMAX), jnp.float32),
        pltpu.VMEM((CHL,), jnp.float32),
        pltpu.VMEM((L,), jnp.float32),
        pltpu.SemaphoreType.DMA,
        pltpu.SemaphoreType.DMA,
        pltpu.SemaphoreType.DMA,
        pltpu.SemaphoreType.DMA,
        pltpu.SemaphoreType.DMA,
    ],
)
def _sc_gather(tabTf_hbm, tabf_hbm, lse_hbm, in_hbm, tg_hbm,
               outT_hbm, part_hbm,
               idxg_v, idx_v, tgt_v, fidx_v, lse_v,
               slab0, slab1, ob0, ob1, telem_v, acc_v,
               g0, g1, s0, s1, t3):
    _sc_body(tabTf_hbm, tabf_hbm, lse_hbm, in_hbm, tg_hbm,
             outT_hbm, part_hbm,
             idxg_v, idx_v, tgt_v, fidx_v, lse_v,
             slab0, slab1, ob0, ob1, telem_v, acc_v,
             g0, g1, s0, s1, t3)


def kernel(input_sequence, target_sequence, table):
    flat_in = input_sequence.reshape(-1)
    flat_tg = target_sequence.reshape(-1)
    tabTf = jnp.pad(table.T.reshape(-1), (0, 8))
    tabf = jnp.pad(table.reshape(-1), (0, 8))
    lse = _row_lse(table)
    predsT, partials = _sc_gather(tabTf, tabf, lse.reshape(-1),
                                  flat_in, flat_tg)
    loss = _finalize(partials)[0, 0]
    return predsT.T, loss


# speedup vs baseline: 2.2503x; 2.2503x over previous
import functools

import jax
import jax.numpy as jnp
from jax import lax
from jax.experimental import pallas as pl
from jax.experimental.pallas import tpu as pltpu
from jax.experimental.pallas import tpu_sc as plsc

V = 1000
N = 51200
NC, NS, L = 2, 16, 16
NW = NC * NS
RPW = N // NW
CHL = 64
NCHL = RPW // CHL
TW0, TW1 = 1536, 1664
TWMAX = TW1
NSLAB = V // 8


def _lse_body(tab_ref, out_ref):
    x = tab_ref[...]
    m = jnp.max(x, axis=1, keepdims=True)
    s = jnp.sum(jnp.exp(x - m), axis=1, keepdims=True)
    out_ref[...] = jnp.log(s) + m


def _row_lse(table):
    return pl.pallas_call(
        _lse_body,
        out_shape=jax.ShapeDtypeStruct((V, 1), jnp.float32),
    )(table)


def _fin_body(p_ref, out_ref):
    out_ref[...] = jnp.full((1, 1), jnp.sum(p_ref[...]) * (1.0 / N),
                            dtype=jnp.float32)


def _finalize(partials):
    return pl.pallas_call(
        _fin_body,
        out_shape=jax.ShapeDtypeStruct((1, 1), jnp.float32),
    )(partials)


def _sc_body(tabTf_hbm, tabf_hbm, lse_hbm, in_hbm, tg_hbm,
             outT_hbm, part_hbm,
             idxg_v, idx_v, tgt_v, fidx_v, lse_v,
             slab0, slab1, ob0, ob1, telem_v, acc_v,
             g0, g1, s0, s1, t3):
    wid = lax.axis_index("s") * NC + lax.axis_index("c")

    base = wid * RPW
    pltpu.sync_copy(in_hbm.at[pl.ds(base, RPW)], idx_v)
    pltpu.sync_copy(tg_hbm.at[pl.ds(base, RPW)], tgt_v)
    pltpu.sync_copy(lse_hbm, lse_v)
    acc_v[...] = jnp.zeros((L,), jnp.float32)

    def fx(i, carry):
        o = i * L
        fidx_v[pl.ds(o, L)] = idx_v[pl.ds(o, L)] * V + tgt_v[pl.ds(o, L)]
        return carry

    lax.fori_loop(0, RPW // L, fx, 0)

    def lchunk(c, carry):
        co = c * CHL
        pltpu.async_copy(
            tabf_hbm.at[fidx_v.at[pl.ds(co, CHL)]], telem_v, t3).wait()
        for g in range(CHL // L):
            ids = idx_v[pl.ds(co + g * L, L)]
            lse_g = plsc.load_gather(lse_v, [ids])
            acc_v[...] = acc_v[...] + (lse_g - telem_v[pl.ds(g * L, L)])
        return carry

    lax.fori_loop(0, NCHL, lchunk, 0)
    pltpu.sync_copy(acc_v, part_hbm.at[wid])

    c0 = jnp.where(wid < 16, wid * TW0, 16 * TW0 + (wid - 16) * TW1)
    pltpu.sync_copy(in_hbm.at[pl.ds(c0, TWMAX)], idxg_v)

    def span_loop(tw):
        nt = tw // 128
        bufs = ((slab0, ob0, g0, s0), (slab1, ob1, g1, s1))

        def fetch(s, slab, gsem):
            pltpu.async_copy(tabTf_hbm.at[pl.ds(s * 8000, 8000)],
                             slab, gsem)

        def store(s, ob, ssem):
            return pltpu.async_copy(
                ob.at[:, pl.ds(0, tw)],
                outT_hbm.at[pl.ds(s * 8, 8), pl.ds(c0, tw)], ssem)

        fetch(0, slab0, g0)
        fetch(1, slab1, g1)

        def do_slab(s, slab, ob, gsem, ssem, first):
            pltpu.make_async_copy(
                tabTf_hbm.at[pl.ds(0, 8000)], slab, gsem).wait()
            if not first:
                pltpu.make_async_copy(
                    ob.at[:, pl.ds(0, tw)],
                    outT_hbm.at[pl.ds(0, 8), pl.ds(c0, tw)], ssem).wait()

            @plsc.parallel_loop(0, nt, 1, unroll=1)
            def _tiles(t):
                for k in range(8):
                    off = t * 128 + k * 16
                    ids = idxg_v[pl.ds(off, L)]
                    vals = [plsc.load_gather(slab, [ids + vv * V])
                            for vv in range(8)]
                    for vv in range(8):
                        ob[vv, pl.ds(off, L)] = vals[vv]

            store(s, ob, ssem)

            @pl.when(s + 2 < NSLAB)
            def _():
                fetch(s + 2, slab, gsem)

        def pair(s2, carry):
            s = s2 * 2
            do_slab(s, slab0, ob0, g0, s0, False)
            do_slab(s + 1, slab1, ob1, g1, s1, False)
            return carry

        do_slab(0, slab0, ob0, g0, s0, True)
        do_slab(1, slab1, ob1, g1, s1, True)
        lax.fori_loop(1, (NSLAB - 1) // 2, pair, 0)
        do_slab(NSLAB - 1, slab0, ob0, g0, s0, False)
        pltpu.make_async_copy(
            ob0.at[:, pl.ds(0, tw)],
            outT_hbm.at[pl.ds(0, 8), pl.ds(c0, tw)], s0).wait()
        pltpu.make_async_copy(
            ob1.at[:, pl.ds(0, tw)],
            outT_hbm.at[pl.ds(0, 8), pl.ds(c0, tw)], s1).wait()

    @pl.when(wid < 16)
    def _():
        span_loop(TW0)

    @pl.when(wid >= 16)
    def _():
        span_loop(TW1)


@functools.partial(
    pl.kernel,
    out_type=(
        jax.ShapeDtypeStruct((V, N), jnp.float32),
        jax.ShapeDtypeStruct((NW, L), jnp.float32),
    ),
    mesh=plsc.VectorSubcoreMesh(core_axis_name="c", subcore_axis_name="s"),
    compiler_params=pltpu.CompilerParams(use_tc_tiling_on_sc=True,
                                         needs_layout_passes=False),
    scratch_types=[
        pltpu.VMEM((TWMAX,), jnp.int32),
        pltpu.VMEM((RPW,), jnp.int32),
        pltpu.VMEM((RPW,), jnp.int32),
        pltpu.VMEM((RPW,), jnp.int32),
        pltpu.VMEM((V,), jnp.float32),
        pltpu.VMEM((8000,), jnp.float32),
        pltpu.VMEM((8000,), jnp.float32),
        pltpu.VMEM((8, TWMAX), jnp.float32),
        pltpu.VMEM((8, TWMAX), jnp.float32),
        pltpu.VMEM((CHL,), jnp.float32),
        pltpu.VMEM((L,), jnp.float32),
        pltpu.SemaphoreType.DMA,
        pltpu.SemaphoreType.DMA,
        pltpu.SemaphoreType.DMA,
        pltpu.SemaphoreType.DMA,
        pltpu.SemaphoreType.DMA,
    ],
)
def _sc_gather(tabTf_hbm, tabf_hbm, lse_hbm, in_hbm, tg_hbm,
               outT_hbm, part_hbm,
               idxg_v, idx_v, tgt_v, fidx_v, lse_v,
               slab0, slab1, ob0, ob1, telem_v, acc_v,
               g0, g1, s0, s1, t3):
    _sc_body(tabTf_hbm, tabf_hbm, lse_hbm, in_hbm, tg_hbm,
             outT_hbm, part_hbm,
             idxg_v, idx_v, tgt_v, fidx_v, lse_v,
             slab0, slab1, ob0, ob1, telem_v, acc_v,
             g0, g1, s0, s1, t3)


def kernel(input_sequence, target_sequence, table):
    flat_in = input_sequence.reshape(-1)
    flat_tg = target_sequence.reshape(-1)
    tabTf = jnp.pad(table.T.reshape(-1), (0, 8))
    tabf = jnp.pad(table.reshape(-1), (0, 8))
    lse = _row_lse(table)
    predsT, partials = _sc_gather(tabTf, tabf, lse.reshape(-1),
                                  flat_in, flat_tg)
    loss = _finalize(partials)[0, 0]
    return predsT.T, loss


# speedup vs baseline: 2.3911x; 1.0626x over previous
import functools

import jax
import jax.numpy as jnp
from jax import lax
from jax.experimental import pallas as pl
from jax.experimental.pallas import tpu as pltpu
from jax.experimental.pallas import tpu_sc as plsc

V = 1000
N = 51200
NC, NS, L = 2, 16, 16
NW = NC * NS
RPW = N // NW
CHL = 64
NCHL = RPW // CHL
TW = N // 8
TH = TW // 2
NTT = TH // 128


def _lse_body(tab_ref, out_ref):
    x = tab_ref[...]
    m = jnp.max(x, axis=1, keepdims=True)
    s = jnp.sum(jnp.exp(x - m), axis=1, keepdims=True)
    out_ref[...] = jnp.log(s) + m


def _row_lse(table):
    return pl.pallas_call(
        _lse_body,
        out_shape=jax.ShapeDtypeStruct((V, 1), jnp.float32),
    )(table)


def _fin_body(p_ref, out_ref):
    out_ref[...] = jnp.full((1, 1), jnp.sum(p_ref[...]) * (1.0 / N),
                            dtype=jnp.float32)


def _finalize(partials):
    return pl.pallas_call(
        _fin_body,
        out_shape=jax.ShapeDtypeStruct((1, 1), jnp.float32),
    )(partials)


def _sc_body(tabTf_hbm, tabf_hbm, lse_hbm, in_hbm, tg_hbm,
             outT_hbm, part_hbm,
             idxg_v, idx_v, tgt_v, fidx_v, lse_v,
             slab0, slab1, ob0, ob1, telem_v, acc_v,
             g0, g1, s0, s1, t3):
    wid = lax.axis_index("s") * NC + lax.axis_index("c")

    base = wid * RPW
    pltpu.sync_copy(in_hbm.at[pl.ds(base, RPW)], idx_v)
    pltpu.sync_copy(tg_hbm.at[pl.ds(base, RPW)], tgt_v)
    pltpu.sync_copy(lse_hbm, lse_v)
    acc_v[...] = jnp.zeros((L,), jnp.float32)

    def fx(i, carry):
        o = i * L
        fidx_v[pl.ds(o, L)] = idx_v[pl.ds(o, L)] * V + tgt_v[pl.ds(o, L)]
        return carry

    lax.fori_loop(0, RPW // L, fx, 0)

    for c in range(NCHL):
        pltpu.async_copy(tabf_hbm.at[fidx_v.at[pl.ds(c * CHL, CHL)]],
                         telem_v.at[pl.ds(c * CHL, CHL)], t3)
    for c in range(NCHL):
        pltpu.make_async_copy(tabf_hbm.at[fidx_v.at[pl.ds(0, CHL)]],
                              telem_v.at[pl.ds(0, CHL)], t3).wait()

    def lgrp(i, carry):
        o = i * L
        lse_g = plsc.load_gather(lse_v, [idx_v[pl.ds(o, L)]])
        acc_v[...] = acc_v[...] + (lse_g - telem_v[pl.ds(o, L)])
        return carry

    lax.fori_loop(0, RPW // L, lgrp, 0)
    pltpu.sync_copy(acc_v, part_hbm.at[wid])

    vg = wid // 8
    v0 = vg * 256
    ns = jnp.where(vg < 3, 32, 29)
    c0 = (wid % 8) * TW
    pltpu.sync_copy(in_hbm.at[pl.ds(c0, TW)], idxg_v)

    def fetch(s, slab, gsem):
        pltpu.async_copy(
            tabTf_hbm.at[pl.ds(v0 * V + s * 8000, 8000)], slab, gsem)

    def process(s, slab, gsem):
        pltpu.make_async_copy(
            tabTf_hbm.at[pl.ds(0, 8000)], slab, gsem).wait()
        for h, (ob, ssem) in enumerate(((ob0, s0), (ob1, s1))):
            @pl.when(s >= 1)
            def _():
                pltpu.make_async_copy(
                    ob, outT_hbm.at[pl.ds(0, 8), pl.ds(0, TH)], ssem
                ).wait()

            @plsc.parallel_loop(0, NTT, 1, unroll=2)
            def _tiles(t):
                for k in range(8):
                    off = h * TH + t * 128 + k * 16
                    ids = idxg_v[pl.ds(off, L)]
                    vals = [plsc.load_gather(slab, [ids + vv * V])
                            for vv in range(8)]
                    for vv in range(8):
                        ob[vv, pl.ds(t * 128 + k * 16, L)] = vals[vv]

            pltpu.async_copy(
                ob,
                outT_hbm.at[pl.ds(v0 + s * 8, 8),
                            pl.ds(c0 + h * TH, TH)], ssem)

        @pl.when(s + 2 < ns)
        def _():
            fetch(s + 2, slab, gsem)

    fetch(0, slab0, g0)
    fetch(1, slab1, g1)

    def slab_loop(s, carry):
        @pl.when(s % 2 == 0)
        def _():
            process(s, slab0, g0)

        @pl.when(s % 2 == 1)
        def _():
            process(s, slab1, g1)
        return carry

    lax.fori_loop(0, ns, slab_loop, 0)
    pltpu.make_async_copy(
        ob0, outT_hbm.at[pl.ds(0, 8), pl.ds(0, TH)], s0).wait()
    pltpu.make_async_copy(
        ob1, outT_hbm.at[pl.ds(0, 8), pl.ds(0, TH)], s1).wait()


@functools.partial(
    pl.kernel,
    out_type=(
        jax.ShapeDtypeStruct((V, N), jnp.float32),
        jax.ShapeDtypeStruct((NW, L), jnp.float32),
    ),
    mesh=plsc.VectorSubcoreMesh(core_axis_name="c", subcore_axis_name="s"),
    compiler_params=pltpu.CompilerParams(use_tc_tiling_on_sc=True,
                                         needs_layout_passes=False),
    scratch_types=[
        pltpu.VMEM((TW,), jnp.int32),
        pltpu.VMEM((RPW,), jnp.int32),
        pltpu.VMEM((RPW,), jnp.int32),
        pltpu.VMEM((RPW,), jnp.int32),
        pltpu.VMEM((V,), jnp.float32),
        pltpu.VMEM((8000,), jnp.float32),
        pltpu.VMEM((8000,), jnp.float32),
        pltpu.VMEM((8, TH), jnp.float32),
        pltpu.VMEM((8, TH), jnp.float32),
        pltpu.VMEM((RPW,), jnp.float32),
        pltpu.VMEM((L,), jnp.float32),
        pltpu.SemaphoreType.DMA,
        pltpu.SemaphoreType.DMA,
        pltpu.SemaphoreType.DMA,
        pltpu.SemaphoreType.DMA,
        pltpu.SemaphoreType.DMA,
    ],
)
def _sc_gather(tabTf_hbm, tabf_hbm, lse_hbm, in_hbm, tg_hbm,
               outT_hbm, part_hbm,
               idxg_v, idx_v, tgt_v, fidx_v, lse_v,
               slab0, slab1, ob0, ob1, telem_v, acc_v,
               g0, g1, s0, s1, t3):
    _sc_body(tabTf_hbm, tabf_hbm, lse_hbm, in_hbm, tg_hbm,
             outT_hbm, part_hbm,
             idxg_v, idx_v, tgt_v, fidx_v, lse_v,
             slab0, slab1, ob0, ob1, telem_v, acc_v,
             g0, g1, s0, s1, t3)


def kernel(input_sequence, target_sequence, table):
    flat_in = input_sequence.reshape(-1)
    flat_tg = target_sequence.reshape(-1)
    tabTf = jnp.pad(table.T.reshape(-1), (0, 8))
    tabf = jnp.pad(table.reshape(-1), (0, 8))
    lse = _row_lse(table)
    predsT, partials = _sc_gather(tabTf, tabf, lse.reshape(-1),
                                  flat_in, flat_tg)
    loss = _finalize(partials)[0, 0]
    return predsT.T, loss


# speedup vs baseline: 2.4167x; 1.0107x over previous
import functools

import jax
import jax.numpy as jnp
from jax import lax
from jax.experimental import pallas as pl
from jax.experimental.pallas import tpu as pltpu
from jax.experimental.pallas import tpu_sc as plsc

V = 1000
N = 51200
NC, NS, L = 2, 16, 16
NW = NC * NS
RPW = N // NW
CHL = 64
NCHL = RPW // CHL
TW = N // 8
TH = TW // 2
NTT = TH // 128


def _lse_body(tab_ref, out_ref):
    x = tab_ref[...]
    m = jnp.max(x, axis=1, keepdims=True)
    s = jnp.sum(jnp.exp(x - m), axis=1, keepdims=True)
    out_ref[...] = jnp.log(s) + m


def _row_lse(table):
    return pl.pallas_call(
        _lse_body,
        out_shape=jax.ShapeDtypeStruct((V, 1), jnp.float32),
    )(table)


def _fin_body(p_ref, out_ref):
    out_ref[...] = jnp.full((1, 1), jnp.sum(p_ref[...]) * (1.0 / N),
                            dtype=jnp.float32)


def _finalize(partials):
    return pl.pallas_call(
        _fin_body,
        out_shape=jax.ShapeDtypeStruct((1, 1), jnp.float32),
    )(partials)


def _sc_body(tabTf_hbm, tabf_hbm, lse_hbm, in_hbm, tg_hbm,
             outT_hbm, part_hbm,
             idxg_v, idx_v, fidx_v, lse_v,
             slab0, slab1, ob0, ob1, ob2, ob3, telem_v, acc_v,
             g0, g1, s0, s1, s2, s3, t3):
    wid = lax.axis_index("s") * NC + lax.axis_index("c")

    base = wid * RPW
    pltpu.sync_copy(in_hbm.at[pl.ds(base, RPW)], idx_v)
    pltpu.sync_copy(tg_hbm.at[pl.ds(base, RPW)], fidx_v)
    pltpu.sync_copy(lse_hbm, lse_v)
    acc_v[...] = jnp.zeros((L,), jnp.float32)

    def fx(i, carry):
        o = i * L
        fidx_v[pl.ds(o, L)] = idx_v[pl.ds(o, L)] * V + fidx_v[pl.ds(o, L)]
        return carry

    lax.fori_loop(0, RPW // L, fx, 0)

    for c in range(NCHL):
        pltpu.async_copy(tabf_hbm.at[fidx_v.at[pl.ds(c * CHL, CHL)]],
                         telem_v.at[pl.ds(c * CHL, CHL)], t3)
    for c in range(NCHL):
        pltpu.make_async_copy(tabf_hbm.at[fidx_v.at[pl.ds(0, CHL)]],
                              telem_v.at[pl.ds(0, CHL)], t3).wait()

    def lgrp(i, carry):
        o = i * L
        lse_g = plsc.load_gather(lse_v, [idx_v[pl.ds(o, L)]])
        acc_v[...] = acc_v[...] + (lse_g - telem_v[pl.ds(o, L)])
        return carry

    lax.fori_loop(0, RPW // L, lgrp, 0)
    pltpu.sync_copy(acc_v, part_hbm.at[wid])

    vg = wid // 8
    v0 = vg * 256
    ns = jnp.where(vg < 3, 32, 29)
    c0 = (wid % 8) * TW
    pltpu.sync_copy(in_hbm.at[pl.ds(c0, TW)], idxg_v)

    def fetch(s, slab, gsem):
        pltpu.async_copy(
            tabTf_hbm.at[pl.ds(v0 * V + s * 8000, 8000)], slab, gsem)

    def process(s, slab, gsem, obufs):
        pltpu.make_async_copy(
            tabTf_hbm.at[pl.ds(0, 8000)], slab, gsem).wait()
        for h, (ob, ssem) in enumerate(obufs):
            @pl.when(s >= 2)
            def _():
                pltpu.make_async_copy(
                    ob, outT_hbm.at[pl.ds(0, 8), pl.ds(0, TH)], ssem
                ).wait()

            @plsc.parallel_loop(0, NTT, 1, unroll=2)
            def _tiles(t):
                for k in range(8):
                    off = h * TH + t * 128 + k * 16
                    ids = idxg_v[pl.ds(off, L)]
                    vals = [plsc.load_gather(slab, [ids + vv * V])
                            for vv in range(8)]
                    for vv in range(8):
                        ob[vv, pl.ds(t * 128 + k * 16, L)] = vals[vv]

            pltpu.async_copy(
                ob,
                outT_hbm.at[pl.ds(v0 + s * 8, 8),
                            pl.ds(c0 + h * TH, TH)], ssem)

        @pl.when(s + 2 < ns)
        def _():
            fetch(s + 2, slab, gsem)

    fetch(0, slab0, g0)
    fetch(1, slab1, g1)

    def slab_loop(s, carry):
        @pl.when(s % 2 == 0)
        def _():
            process(s, slab0, g0, ((ob0, s0), (ob1, s1)))

        @pl.when(s % 2 == 1)
        def _():
            process(s, slab1, g1, ((ob2, s2), (ob3, s3)))
        return carry

    lax.fori_loop(0, ns, slab_loop, 0)
    for ob, ssem in ((ob0, s0), (ob1, s1), (ob2, s2), (ob3, s3)):
        pltpu.make_async_copy(
            ob, outT_hbm.at[pl.ds(0, 8), pl.ds(0, TH)], ssem).wait()


@functools.partial(
    pl.kernel,
    out_type=(
        jax.ShapeDtypeStruct((V, N), jnp.float32),
        jax.ShapeDtypeStruct((NW, L), jnp.float32),
    ),
    mesh=plsc.VectorSubcoreMesh(core_axis_name="c", subcore_axis_name="s"),
    compiler_params=pltpu.CompilerParams(use_tc_tiling_on_sc=True,
                                         needs_layout_passes=False),
    scratch_types=[
        pltpu.VMEM((TW,), jnp.int32),
        pltpu.VMEM((RPW,), jnp.int32),
        pltpu.VMEM((RPW,), jnp.int32),
        pltpu.VMEM((V,), jnp.float32),
        pltpu.VMEM((8000,), jnp.float32),
        pltpu.VMEM((8000,), jnp.float32),
        pltpu.VMEM((8, TH), jnp.float32),
        pltpu.VMEM((8, TH), jnp.float32),
        pltpu.VMEM((8, TH), jnp.float32),
        pltpu.VMEM((8, TH), jnp.float32),
        pltpu.VMEM((RPW,), jnp.float32),
        pltpu.VMEM((L,), jnp.float32),
        pltpu.SemaphoreType.DMA,
        pltpu.SemaphoreType.DMA,
        pltpu.SemaphoreType.DMA,
        pltpu.SemaphoreType.DMA,
        pltpu.SemaphoreType.DMA,
        pltpu.SemaphoreType.DMA,
        pltpu.SemaphoreType.DMA,
    ],
)
def _sc_gather(tabTf_hbm, tabf_hbm, lse_hbm, in_hbm, tg_hbm,
               outT_hbm, part_hbm,
               idxg_v, idx_v, fidx_v, lse_v,
               slab0, slab1, ob0, ob1, ob2, ob3, telem_v, acc_v,
               g0, g1, s0, s1, s2, s3, t3):
    _sc_body(tabTf_hbm, tabf_hbm, lse_hbm, in_hbm, tg_hbm,
             outT_hbm, part_hbm,
             idxg_v, idx_v, fidx_v, lse_v,
             slab0, slab1, ob0, ob1, ob2, ob3, telem_v, acc_v,
             g0, g1, s0, s1, s2, s3, t3)


def kernel(input_sequence, target_sequence, table):
    flat_in = input_sequence.reshape(-1)
    flat_tg = target_sequence.reshape(-1)
    tabTf = jnp.pad(table.T.reshape(-1), (0, 8))
    tabf = jnp.pad(table.reshape(-1), (0, 8))
    lse = _row_lse(table)
    predsT, partials = _sc_gather(tabTf, tabf, lse.reshape(-1),
                                  flat_in, flat_tg)
    loss = _finalize(partials)[0, 0]
    return predsT.T, loss


# speedup vs baseline: 3.2999x; 1.3655x over previous
import functools

import jax
import jax.numpy as jnp
from jax import lax
from jax.experimental import pallas as pl
from jax.experimental.pallas import tpu as pltpu
from jax.experimental.pallas import tpu_sc as plsc

V = 1000
N = 51200
NC, NS, L = 2, 16, 16
NW = NC * NS
RPW = N // NW
CHL = 64
NCHL = RPW // CHL
TW = N // 8
TH = TW // 2
NTT = TH // 128


def _lse_body(tab_ref, out_ref):
    x = tab_ref[...]
    m = jnp.max(x, axis=1, keepdims=True)
    s = jnp.sum(jnp.exp(x - m), axis=1, keepdims=True)
    out_ref[...] = jnp.log(s) + m


def _row_lse(table):
    return pl.pallas_call(
        _lse_body,
        out_shape=jax.ShapeDtypeStruct((V, 1), jnp.float32),
    )(table)


def _fin_body(p_ref, out_ref):
    out_ref[...] = jnp.full((1, 1), jnp.sum(p_ref[...]) * (1.0 / N),
                            dtype=jnp.float32)


def _finalize(partials):
    return pl.pallas_call(
        _fin_body,
        out_shape=jax.ShapeDtypeStruct((1, 1), jnp.float32),
    )(partials)


def _sc_body(tabTf_hbm, tabf_hbm, lse_hbm, in_hbm, tg_hbm,
             outT_hbm, part_hbm,
             idxg_v, idx_v, fidx_v, lse_v,
             slab0, slab1, ob0, ob1, ob2, ob3, telem_v, acc_v,
             g0, g1, s0, s1, s2, s3, t3):
    wid = lax.axis_index("s") * NC + lax.axis_index("c")

    base = wid * RPW
    pltpu.sync_copy(in_hbm.at[pl.ds(base, RPW)], idx_v)
    pltpu.sync_copy(tg_hbm.at[pl.ds(base, RPW)], fidx_v)
    pltpu.sync_copy(lse_hbm, lse_v)
    acc_v[...] = jnp.zeros((L,), jnp.float32)

    def fx(i, carry):
        o = i * L
        fidx_v[pl.ds(o, L)] = idx_v[pl.ds(o, L)] * V + fidx_v[pl.ds(o, L)]
        return carry

    lax.fori_loop(0, RPW // L, fx, 0)

    for c in range(NCHL):
        pltpu.async_copy(tabf_hbm.at[fidx_v.at[pl.ds(c * CHL, CHL)]],
                         telem_v.at[pl.ds(c * CHL, CHL)], t3)
    for c in range(NCHL):
        pltpu.make_async_copy(tabf_hbm.at[fidx_v.at[pl.ds(0, CHL)]],
                              telem_v.at[pl.ds(0, CHL)], t3).wait()

    def lgrp(i, carry):
        o = i * L
        lse_g = plsc.load_gather(lse_v, [idx_v[pl.ds(o, L)]])
        acc_v[...] = acc_v[...] + (lse_g - telem_v[pl.ds(o, L)])
        return carry

    lax.fori_loop(0, RPW // L, lgrp, 0)
    pltpu.sync_copy(acc_v, part_hbm.at[wid])

    vg = wid // 8
    v0 = vg * 256
    ns = jnp.where(vg < 3, 32, 29)
    c0 = (wid % 8) * TW
    pltpu.sync_copy(in_hbm.at[pl.ds(c0, TW)], idxg_v)

    def fetch(s, slab, gsem):
        pltpu.async_copy(
            tabTf_hbm.at[pl.ds(v0 * V + s * 8000, 8000)], slab, gsem)

    def process(s, slab, gsem, obufs):
        pltpu.make_async_copy(
            tabTf_hbm.at[pl.ds(0, 8000)], slab, gsem).wait()
        for h, (ob, ssem) in enumerate(obufs):
            @pl.when(s >= 2)
            def _():
                pltpu.make_async_copy(
                    ob, outT_hbm.at[pl.ds(0, 8), pl.ds(0, TH)], ssem
                ).wait()

            @plsc.parallel_loop(0, NTT, 1, unroll=4)
            def _tiles(t):
                for k in range(8):
                    off = h * TH + t * 128 + k * 16
                    ids = idxg_v[pl.ds(off, L)]
                    vals = [plsc.load_gather(slab, [ids + vv * V])
                            for vv in range(8)]
                    for vv in range(8):
                        ob[vv, pl.ds(t * 128 + k * 16, L)] = vals[vv]

            pltpu.async_copy(
                ob,
                outT_hbm.at[pl.ds(v0 + s * 8, 8),
                            pl.ds(c0 + h * TH, TH)], ssem)

        @pl.when(s + 2 < ns)
        def _():
            fetch(s + 2, slab, gsem)

    fetch(0, slab0, g0)
    fetch(1, slab1, g1)

    def slab_loop(s, carry):
        @pl.when(s % 2 == 0)
        def _():
            process(s, slab0, g0, ((ob0, s0), (ob1, s1)))

        @pl.when(s % 2 == 1)
        def _():
            process(s, slab1, g1, ((ob2, s2), (ob3, s3)))
        return carry

    lax.fori_loop(0, ns, slab_loop, 0)
    for ob, ssem in ((ob0, s0), (ob1, s1), (ob2, s2), (ob3, s3)):
        pltpu.make_async_copy(
            ob, outT_hbm.at[pl.ds(0, 8), pl.ds(0, TH)], ssem).wait()


@functools.partial(
    pl.kernel,
    out_type=(
        jax.ShapeDtypeStruct((V, N), jnp.float32),
        jax.ShapeDtypeStruct((NW, L), jnp.float32),
    ),
    mesh=plsc.VectorSubcoreMesh(core_axis_name="c", subcore_axis_name="s"),
    compiler_params=pltpu.CompilerParams(use_tc_tiling_on_sc=True,
                                         needs_layout_passes=False),
    scratch_types=[
        pltpu.VMEM((TW,), jnp.int32),
        pltpu.VMEM((RPW,), jnp.int32),
        pltpu.VMEM((RPW,), jnp.int32),
        pltpu.VMEM((V,), jnp.float32),
        pltpu.VMEM((8000,), jnp.float32),
        pltpu.VMEM((8000,), jnp.float32),
        pltpu.VMEM((8, TH), jnp.float32),
        pltpu.VMEM((8, TH), jnp.float32),
        pltpu.VMEM((8, TH), jnp.float32),
        pltpu.VMEM((8, TH), jnp.float32),
        pltpu.VMEM((RPW,), jnp.float32),
        pltpu.VMEM((L,), jnp.float32),
        pltpu.SemaphoreType.DMA,
        pltpu.SemaphoreType.DMA,
        pltpu.SemaphoreType.DMA,
        pltpu.SemaphoreType.DMA,
        pltpu.SemaphoreType.DMA,
        pltpu.SemaphoreType.DMA,
        pltpu.SemaphoreType.DMA,
    ],
)
def _sc_gather(tabTf_hbm, tabf_hbm, lse_hbm, in_hbm, tg_hbm,
               outT_hbm, part_hbm,
               idxg_v, idx_v, fidx_v, lse_v,
               slab0, slab1, ob0, ob1, ob2, ob3, telem_v, acc_v,
               g0, g1, s0, s1, s2, s3, t3):
    _sc_body(tabTf_hbm, tabf_hbm, lse_hbm, in_hbm, tg_hbm,
             outT_hbm, part_hbm,
             idxg_v, idx_v, fidx_v, lse_v,
             slab0, slab1, ob0, ob1, ob2, ob3, telem_v, acc_v,
             g0, g1, s0, s1, s2, s3, t3)


def kernel(input_sequence, target_sequence, table):
    flat_in = input_sequence.reshape(-1)
    flat_tg = target_sequence.reshape(-1)
    tabTf = jnp.pad(table.T.reshape(-1), (0, 8))
    tabf = jnp.pad(table.reshape(-1), (0, 8))
    lse = _row_lse(table)
    predsT, partials = _sc_gather(tabTf, tabf, lse.reshape(-1),
                                  flat_in, flat_tg)
    loss = _finalize(partials)[0, 0]
    return predsT.T, loss
